# baseline (speedup 1.0000x reference)
import functools

import jax
import jax.numpy as jnp
from jax import lax
from jax.experimental import pallas as pl
from jax.experimental.pallas import tpu as pltpu
from jax.experimental.pallas import tpu_sc as plsc

MAXD = 128
NT = 2 * MAXD + 1
SEQ = 512
DIM = 128

_NC = 2
_NS = 16
_NW = _NC * _NS
_RPW = SEQ // _NW


def _table_body(dist_ref, dir_ref, w_ref, b_ref, out_ref):
    dist = dist_ref[...]
    wt = w_ref[0:DIM, :]
    wb = w_ref[DIM:2 * DIM, :]
    t_dist = jnp.dot(dist, wt, preferred_element_type=jnp.float32)
    dv = jnp.dot(dir_ref[...], wb, preferred_element_type=jnp.float32)
    k = lax.broadcasted_iota(jnp.int32, (NT, 1), 0)
    dcon = jnp.where(k < MAXD, dv[0:1, :],
                     jnp.where(k == MAXD, dv[1:2, :], dv[2:3, :]))
    t = t_dist + dcon + b_ref[...]

    m = lax.broadcasted_iota(jnp.int32, (1024, NT), 0)
    kk = lax.broadcasted_iota(jnp.int32, (1024, NT), 1)
    idx = jnp.clip(639 - m, 0, NT - 1)
    onehot = (kk == idx).astype(jnp.float32)
    out_ref[...] = jnp.dot(onehot, t, preferred_element_type=jnp.float32)


def _make_hrev(dist_table, dir_table, fusion_W, fusion_b2d):
    return pl.pallas_call(
        _table_body,
        out_shape=jax.ShapeDtypeStruct((1024, DIM), jnp.float32),
    )(dist_table, dir_table, fusion_W, fusion_b2d)


@functools.cache
def _make_broadcast():
    @functools.partial(
        pl.kernel,
        out_type=jax.ShapeDtypeStruct((SEQ, SEQ, DIM), jnp.float32),
        mesh=plsc.VectorSubcoreMesh(core_axis_name="c", subcore_axis_name="s"),
        scratch_types=[
            pltpu.VMEM_SHARED((1024, DIM), jnp.float32),
            pltpu.SemaphoreType.DMA,
        ],
    )
    def _broadcast(hrev_hbm, out_hbm, shared, sem):
        c = lax.axis_index("c")
        s = lax.axis_index("s")

        @pl.when(s == 0)
        def _stage():
            pltpu.sync_copy(hrev_hbm, shared)

        plsc.subcore_barrier()
        base = (s * _NC + c) * _RPW
        copies = [
            pltpu.async_copy(shared.at[pl.ds(511 - base - t, SEQ)],
                             out_hbm.at[base + t], sem)
            for t in range(_RPW)
        ]
        for cpy in copies:
            cpy.wait()

    return _broadcast


def kernel(x, dist_table, dir_table, fusion_W, fusion_b):
    hrev = _make_hrev(dist_table, dir_table, fusion_W,
                      fusion_b.reshape(1, DIM))
    out = _make_broadcast()(hrev)
    return out.reshape(1, SEQ, SEQ, DIM)


# speedup vs baseline: 1.3433x; 1.3433x over previous
import functools

import jax
import jax.numpy as jnp
from jax import lax
from jax.experimental import pallas as pl
from jax.experimental.pallas import tpu as pltpu
from jax.experimental.pallas import tpu_sc as plsc

MAXD = 128
NT = 2 * MAXD + 1
SEQ = 512
DIM = 128

_NC = 2
_NS = 16
_NW = _NC * _NS
_RPW = SEQ // _NW
_STAGE = 528


def _table_body(dist_ref, dir_ref, w_ref, b_ref, out_ref):
    dist = dist_ref[...]
    wt = w_ref[0:DIM, :]
    wb = w_ref[DIM:2 * DIM, :]
    t_dist = jnp.dot(dist, wt, preferred_element_type=jnp.float32)
    dv = jnp.dot(dir_ref[...], wb, preferred_element_type=jnp.float32)
    k = lax.broadcasted_iota(jnp.int32, (NT, 1), 0)
    dcon = jnp.where(k < MAXD, dv[0:1, :],
                     jnp.where(k == MAXD, dv[1:2, :], dv[2:3, :]))
    t = t_dist + dcon + b_ref[...]

    m = lax.broadcasted_iota(jnp.int32, (1024, NT), 0)
    kk = lax.broadcasted_iota(jnp.int32, (1024, NT), 1)
    idx = jnp.clip(639 - m, 0, NT - 1)
    onehot = (kk == idx).astype(jnp.float32)
    out_ref[...] = jnp.dot(onehot, t, preferred_element_type=jnp.float32)


def _make_hrev(dist_table, dir_table, fusion_W, fusion_b2d):
    return pl.pallas_call(
        _table_body,
        out_shape=jax.ShapeDtypeStruct((1024, DIM), jnp.float32),
    )(dist_table, dir_table, fusion_W, fusion_b2d)


@functools.cache
def _make_broadcast():
    @functools.partial(
        pl.kernel,
        out_type=jax.ShapeDtypeStruct((SEQ, SEQ, DIM), jnp.float32),
        mesh=plsc.VectorSubcoreMesh(core_axis_name="c", subcore_axis_name="s"),
        scratch_types=[
            pltpu.VMEM((_STAGE, DIM), jnp.float32),
            pltpu.SemaphoreType.DMA,
        ],
    )
    def _broadcast(hrev_hbm, out_hbm, stage_v, sem):
        wid = lax.axis_index("s") * _NC + lax.axis_index("c")
        base = wid * _RPW
        pltpu.sync_copy(hrev_hbm.at[pl.ds(496 - base, _STAGE)], stage_v)
        copies = [
            pltpu.async_copy(stage_v.at[pl.ds(_RPW - 1 - t, SEQ)],
                             out_hbm.at[base + t], sem)
            for t in range(_RPW)
        ]
        for c in copies:
            c.wait()

    return _broadcast


def kernel(x, dist_table, dir_table, fusion_W, fusion_b):
    hrev = _make_hrev(dist_table, dir_table, fusion_W,
                      fusion_b.reshape(1, DIM))
    out = _make_broadcast()(hrev)
    return out.reshape(1, SEQ, SEQ, DIM)
